# baseline (speedup 1.0000x reference)
import functools

import jax
import jax.numpy as jnp
from jax import lax
from jax.experimental import pallas as pl
from jax.experimental.pallas import tpu as pltpu
from jax.experimental.pallas import tpu_sc as plsc

EMB_DIM = 64
BATCH = 16384
LANE = 128
NUM_SC = 2
NUM_SUBCORES = 16
NW = NUM_SC * NUM_SUBCORES
B_PER_W = BATCH // NW
GRP = 32
NBUF = 4
VEC = 16
ROW_CHUNK = 128

_sc_mesh = plsc.VectorSubcoreMesh(
    core_axis_name="c", subcore_axis_name="s",
    num_cores=NUM_SC, num_subcores=NUM_SUBCORES,
)


@functools.partial(
    pl.kernel,
    out_type=(
        jax.ShapeDtypeStruct((BATCH, EMB_DIM), jnp.float32),
        jax.ShapeDtypeStruct((BATCH, EMB_DIM), jnp.float32),
    ),
    mesh=_sc_mesh,
    scratch_types=[
        pltpu.VMEM((B_PER_W,), jnp.int32),
        pltpu.VMEM((B_PER_W,), jnp.int32),
    ] + [pltpu.VMEM((EMB_DIM, LANE), jnp.float32) for _ in range(2 * NBUF)] + [
        pltpu.VMEM((ROW_CHUNK, EMB_DIM), jnp.float32),
        pltpu.VMEM((ROW_CHUNK, EMB_DIM), jnp.float32),
        pltpu.SemaphoreType.DMA,
    ],
    compiler_params=pltpu.CompilerParams(needs_layout_passes=False),
)
def _sc_gather(targ_t, cont_t, targ_idx, cont_idx, t_out, c_out,
               ti_v, ci_v, blk_t0, blk_t1, blk_t2, blk_t3,
               blk_c0, blk_c1, blk_c2, blk_c3, t_rows, c_rows, sem):
    wid = lax.axis_index("s") * NUM_SC + lax.axis_index("c")
    base = wid * B_PER_W
    pltpu.sync_copy(targ_idx.at[pl.ds(base, B_PER_W)], ti_v)
    pltpu.sync_copy(cont_idx.at[pl.ds(base, B_PER_W)], ci_v)
    iota16 = lax.iota(jnp.int32, VEC)
    bufs_t = (blk_t0, blk_t1, blk_t2, blk_t3)
    bufs_c = (blk_c0, blk_c1, blk_c2, blk_c3)
    depth = NBUF - 1

    def fetch(table, idx_scalar, blk):
        tb = pl.multiple_of((idx_scalar >> 7) << 7, LANE)
        return pltpu.async_copy(table.at[:, pl.ds(tb, LANE)], blk, sem)

    def extract(blk, idx_scalar, rows, rel):
        col = jnp.broadcast_to(idx_scalar & (LANE - 1), (VEC,))
        for d16 in range(EMB_DIM // VEC):
            dv = iota16 + (d16 * VEC)
            vals = plsc.load_gather(blk, [dv, col])
            rows[rel, pl.ds(d16 * VEC, VEC)] = vals

    def chunk(cc, carry):
        def group(g, carry2):
            r0 = cc * ROW_CHUNK + g * GRP
            tvs = [ti_v[pl.ds(r0 + v * VEC, VEC)] for v in range(GRP // VEC)]
            cvs = [ci_v[pl.ds(r0 + v * VEC, VEC)] for v in range(GRP // VEC)]

            def idx_t(k):
                return tvs[k // VEC][k % VEC]

            def idx_c(k):
                return cvs[k // VEC][k % VEC]

            cps = []
            for k in range(depth):
                cps.append(fetch(targ_t, idx_t(k), bufs_t[k % NBUF]))
                cps.append(fetch(cont_t, idx_c(k), bufs_c[k % NBUF]))
            for k in range(GRP):
                rel = g * GRP + k
                if k + depth < GRP:
                    cps.append(
                        fetch(targ_t, idx_t(k + depth), bufs_t[(k + depth) % NBUF]))
                    cps.append(
                        fetch(cont_t, idx_c(k + depth), bufs_c[(k + depth) % NBUF]))
                cps.pop(0).wait()
                extract(bufs_t[k % NBUF], idx_t(k), t_rows, rel)
                cps.pop(0).wait()
                extract(bufs_c[k % NBUF], idx_c(k), c_rows, rel)
            return carry2

        lax.fori_loop(0, ROW_CHUNK // GRP, group, 0, unroll=False)
        pltpu.sync_copy(t_rows, t_out.at[pl.ds(base + cc * ROW_CHUNK, ROW_CHUNK)])
        pltpu.sync_copy(c_rows, c_out.at[pl.ds(base + cc * ROW_CHUNK, ROW_CHUNK)])
        return carry

    lax.fori_loop(0, B_PER_W // ROW_CHUNK, chunk, 0, unroll=False)


def _loss_body(t_ref, c_ref, out_ref):
    prod = t_ref[...] * c_ref[...]
    score = jnp.sum(prod, axis=1)
    out_ref[...] = jnp.full((1, 1), -jnp.sum(jax.nn.log_sigmoid(score)),
                            dtype=jnp.float32)


_loss_call = pl.pallas_call(
    _loss_body,
    out_shape=jax.ShapeDtypeStruct((1, 1), jnp.float32),
)


@jax.jit
def kernel(targ, cont, targ_table, cont_table):
    ti = targ.astype(jnp.int32)
    ci = cont.astype(jnp.int32)
    perm = jnp.argsort(ti)
    t_ems, c_ems = _sc_gather(targ_table.T, cont_table.T, ti[perm], ci[perm])
    return _loss_call(t_ems, c_ems)[0, 0]


# speedup vs baseline: 1.0515x; 1.0515x over previous
import functools

import jax
import jax.numpy as jnp
from jax import lax
from jax.experimental import pallas as pl
from jax.experimental.pallas import tpu as pltpu
from jax.experimental.pallas import tpu_sc as plsc

EMB_DIM = 64
BATCH = 16384
LANE = 128
NUM_SC = 2
NUM_SUBCORES = 16
NW = NUM_SC * NUM_SUBCORES
B_PER_W = BATCH // NW
GRP = 32
NBUF = 5
VEC = 16
ROW_CHUNK = 128

_sc_mesh = plsc.VectorSubcoreMesh(
    core_axis_name="c", subcore_axis_name="s",
    num_cores=NUM_SC, num_subcores=NUM_SUBCORES,
)


@functools.partial(
    pl.kernel,
    out_type=(
        jax.ShapeDtypeStruct((BATCH, EMB_DIM), jnp.float32),
        jax.ShapeDtypeStruct((BATCH, EMB_DIM), jnp.float32),
    ),
    mesh=_sc_mesh,
    scratch_types=[
        pltpu.VMEM((B_PER_W,), jnp.int32),
        pltpu.VMEM((B_PER_W,), jnp.int32),
    ] + [pltpu.VMEM((EMB_DIM, LANE), jnp.float32) for _ in range(2 * NBUF)] + [
        pltpu.VMEM((ROW_CHUNK, EMB_DIM), jnp.float32),
        pltpu.VMEM((ROW_CHUNK, EMB_DIM), jnp.float32),
        pltpu.SemaphoreType.DMA,
    ],
    compiler_params=pltpu.CompilerParams(needs_layout_passes=False),
)
def _sc_gather(targ_t, cont_t, targ_idx, cont_idx, t_out, c_out,
               ti_v, ci_v, blk_t0, blk_t1, blk_t2, blk_t3, blk_t4,
               blk_c0, blk_c1, blk_c2, blk_c3, blk_c4,
               t_rows, c_rows, sem):
    wid = lax.axis_index("s") * NUM_SC + lax.axis_index("c")
    base = wid * B_PER_W
    pltpu.sync_copy(targ_idx.at[pl.ds(base, B_PER_W)], ti_v)
    pltpu.sync_copy(cont_idx.at[pl.ds(base, B_PER_W)], ci_v)
    iota16 = lax.iota(jnp.int32, VEC)
    bufs_t = (blk_t0, blk_t1, blk_t2, blk_t3, blk_t4)
    bufs_c = (blk_c0, blk_c1, blk_c2, blk_c3, blk_c4)
    depth = NBUF - 1

    def fetch(table, idx_scalar, blk):
        tb = pl.multiple_of((idx_scalar >> 7) << 7, LANE)
        return pltpu.async_copy(table.at[:, pl.ds(tb, LANE)], blk, sem)

    def extract(blk, idx_scalar, rows, rel):
        col = jnp.broadcast_to(idx_scalar & (LANE - 1), (VEC,))
        for d16 in range(EMB_DIM // VEC):
            dv = iota16 + (d16 * VEC)
            vals = plsc.load_gather(blk, [dv, col])
            rows[rel, pl.ds(d16 * VEC, VEC)] = vals

    def chunk(cc, carry):
        def group(g, carry2):
            r0 = cc * ROW_CHUNK + g * GRP
            tvs = [ti_v[pl.ds(r0 + v * VEC, VEC)] for v in range(GRP // VEC)]
            cvs = [ci_v[pl.ds(r0 + v * VEC, VEC)] for v in range(GRP // VEC)]

            def idx_t(k):
                return tvs[k // VEC][k % VEC]

            def idx_c(k):
                return cvs[k // VEC][k % VEC]

            cps = []
            for k in range(depth):
                cps.append(fetch(targ_t, idx_t(k), bufs_t[k % NBUF]))
                cps.append(fetch(cont_t, idx_c(k), bufs_c[k % NBUF]))
            for k in range(GRP):
                rel = g * GRP + k
                if k + depth < GRP:
                    cps.append(
                        fetch(targ_t, idx_t(k + depth), bufs_t[(k + depth) % NBUF]))
                    cps.append(
                        fetch(cont_t, idx_c(k + depth), bufs_c[(k + depth) % NBUF]))
                cps.pop(0).wait()
                extract(bufs_t[k % NBUF], idx_t(k), t_rows, rel)
                cps.pop(0).wait()
                extract(bufs_c[k % NBUF], idx_c(k), c_rows, rel)
            return carry2

        lax.fori_loop(0, ROW_CHUNK // GRP, group, 0, unroll=False)
        pltpu.sync_copy(t_rows, t_out.at[pl.ds(base + cc * ROW_CHUNK, ROW_CHUNK)])
        pltpu.sync_copy(c_rows, c_out.at[pl.ds(base + cc * ROW_CHUNK, ROW_CHUNK)])
        return carry

    lax.fori_loop(0, B_PER_W // ROW_CHUNK, chunk, 0, unroll=False)


def _loss_body(t_ref, c_ref, out_ref):
    prod = t_ref[...] * c_ref[...]
    score = jnp.sum(prod, axis=1)
    out_ref[...] = jnp.full((1, 1), -jnp.sum(jax.nn.log_sigmoid(score)),
                            dtype=jnp.float32)


_loss_call = pl.pallas_call(
    _loss_body,
    out_shape=jax.ShapeDtypeStruct((1, 1), jnp.float32),
)


@jax.jit
def kernel(targ, cont, targ_table, cont_table):
    ti = targ.astype(jnp.int32)
    ci = cont.astype(jnp.int32)
    t_ems, c_ems = _sc_gather(targ_table.T, cont_table.T, ti, ci)
    return _loss_call(t_ems, c_ems)[0, 0]
